# Initial kernel scaffold; baseline (speedup 1.0000x reference)
import functools

import jax
import jax.numpy as jnp
from jax import lax
from jax.experimental import pallas as pl
from jax.experimental.pallas import tpu as pltpu
from jax.experimental.pallas import tpu_sc as plsc

T = 3
N = 10000
D = 128
NP = 10240
NC, NS, LANES = 2, 16, 16
NW = NC * NS
EP = 2 * 320000 + NP
PER_TILE = EP // NW
CHUNK = 80
NCH = PER_TILE // CHUNK
ROWS_PER_TILE = NP // NS
ZR = 80
NBUF = 4
BR = 1024
NB = NP // BR


def _mesh():
  return plsc.VectorSubcoreMesh(
      core_axis_name="c", subcore_axis_name="s",
      num_cores=NC, num_subcores=NS)


def _deg_body(dst_hbm, out_hbm, idx_v, ones_v, zer_v, acc_sh, dsem):
  cid = lax.axis_index("c")
  sid = lax.axis_index("s")
  wid = cid * NS + sid
  zslice = (T * NP) // NS

  for j in range(CHUNK // LANES):
    ones_v[pl.ds(j * LANES, LANES)] = jnp.full((LANES,), 1.0, jnp.float32)

  def zbody(j, c):
    zer_v[pl.ds(j * LANES, LANES)] = jnp.zeros((LANES,), jnp.float32)
    return c
  lax.fori_loop(0, zslice // LANES, zbody, 0)

  pltpu.sync_copy(dst_hbm.at[wid], idx_v)
  pltpu.sync_copy(zer_v, acc_sh.at[pl.ds(sid * zslice, zslice)])
  plsc.subcore_barrier()

  G = 6
  def gbody(g, c):
    for u in range(G):
      pltpu.async_copy(ones_v, acc_sh.at[idx_v.at[g * G + u]], dsem, add=True)
    for _ in range(G):
      pltpu.make_async_copy(ones_v, acc_sh.at[idx_v.at[0]], dsem).wait()
    return c
  lax.fori_loop(0, (T * NCH) // G, gbody, 0)
  plsc.subcore_barrier()

  pltpu.sync_copy(acc_sh.at[pl.ds(sid * zslice, zslice)],
                  out_hbm.at[cid, pl.ds(sid * zslice, zslice)])


def _gs_body(table_hbm, src_hbm, dst_hbm, out_hbm,
             src_v, dst_v, bufs, zrows, acc_sh, gsem, ssem):
  cid = lax.axis_index("c")
  sid = lax.axis_index("s")
  wid = cid * NS + sid
  row0 = sid * ROWS_PER_TILE

  def zbody(k, c):
    zrows[k >> 3, pl.ds((k & 7) * LANES, LANES)] = jnp.zeros(
        (LANES,), jnp.float32)
    return c
  lax.fori_loop(0, ZR * (D // LANES), zbody, 0)

  def start_gather(c, b):
    pltpu.async_copy(table_hbm.at[src_v.at[c]], bufs.at[b], gsem)

  def wait_gather():
    pltpu.make_async_copy(table_hbm.at[src_v.at[0]], bufs.at[0], gsem).wait()

  def start_scatter(c, b):
    pltpu.async_copy(bufs.at[b], acc_sh.at[dst_v.at[c]], ssem, add=True)

  def wait_scatter():
    pltpu.make_async_copy(bufs.at[0], acc_sh.at[dst_v.at[0]], ssem).wait()

  for t in range(T):
    pltpu.sync_copy(src_hbm.at[wid, t], src_v)
    pltpu.sync_copy(dst_hbm.at[wid, t], dst_v)
    if t > 0:
      pltpu.sync_copy(acc_sh.at[pl.ds(row0, ROWS_PER_TILE)],
                      out_hbm.at[cid, t - 1, pl.ds(row0, ROWS_PER_TILE)])
    for z in range(ROWS_PER_TILE // ZR):
      pltpu.sync_copy(zrows, acc_sh.at[pl.ds(row0 + z * ZR, ZR)])
    plsc.subcore_barrier()

    start_gather(0, 0)
    start_gather(1, 1)

    def cbody(c, carry):
      wait_gather()
      start_scatter(c, lax.rem(c, NBUF))

      @pl.when(c >= 2)
      def _():
        wait_scatter()

      @pl.when(c + 2 < NCH)
      def _():
        start_gather(c + 2, lax.rem(c + 2, NBUF))
      return carry
    lax.fori_loop(0, NCH, cbody, 0)
    wait_scatter()
    wait_scatter()
    plsc.subcore_barrier()

  pltpu.sync_copy(acc_sh.at[pl.ds(row0, ROWS_PER_TILE)],
                  out_hbm.at[cid, T - 1, pl.ds(row0, ROWS_PER_TILE)])


@jax.jit
def _sc_degrees(dst_deg):
  return pl.kernel(
      _deg_body,
      out_type=jax.ShapeDtypeStruct((NC, T * NP), jnp.float32),
      mesh=_mesh(),
      scratch_types=[
          pltpu.VMEM((T * NCH, CHUNK), jnp.int32),
          pltpu.VMEM((CHUNK,), jnp.float32),
          pltpu.VMEM(((T * NP) // NS,), jnp.float32),
          pltpu.VMEM_SHARED((T * NP,), jnp.float32),
          pltpu.SemaphoreType.DMA,
      ],
  )(dst_deg)


@jax.jit
def _sc_gather_scatter(table, src_g, dst_l):
  return pl.kernel(
      _gs_body,
      out_type=jax.ShapeDtypeStruct((NC, T, NP, D), jnp.float32),
      mesh=_mesh(),
      scratch_types=[
          pltpu.VMEM((NCH, CHUNK), jnp.int32),
          pltpu.VMEM((NCH, CHUNK), jnp.int32),
          pltpu.VMEM((NBUF, CHUNK, D), jnp.float32),
          pltpu.VMEM((ZR, D), jnp.float32),
          pltpu.VMEM_SHARED((NP, D), jnp.float32),
          pltpu.SemaphoreType.DMA,
          pltpu.SemaphoreType.DMA,
      ],
  )(table, src_g, dst_l)


def _dinv_cols(degb):
  dtot = degb[0:1, :] + degb[1:2, :]
  dinv = lax.rsqrt(dtot)
  eye = jnp.eye(D, dtype=jnp.float32)
  cols = []
  for s in range(BR // D):
    band = dinv[:, s * D:(s + 1) * D]
    cols.append(lax.dot_general(
        eye, band, (((1,), (1,)), ((), ())),
        preferred_element_type=jnp.float32))
  return jnp.concatenate(cols, axis=0)


def _tc1_body(x_ref, wl_ref, w1_ref, deg_ref, o_ref):
  xb = x_ref[0]
  xw = jnp.dot(jnp.dot(xb, wl_ref[...], preferred_element_type=jnp.float32),
               w1_ref[...], preferred_element_type=jnp.float32)
  o_ref[0] = xw * _dinv_cols(deg_ref[...])


def _tc2_body(acc_ref, deg_ref, w2_ref, b1_ref, o_ref):
  a = acc_ref[0, 0] + acc_ref[1, 0]
  dcol = _dinv_cols(deg_ref[...])
  h = jnp.maximum(a * dcol + b1_ref[...], 0.0)
  o_ref[0] = jnp.dot(h, w2_ref[...],
                     preferred_element_type=jnp.float32) * dcol


def _tc3_body(acc_ref, deg_ref, b2_ref, o_ref):
  a = acc_ref[0, 0] + acc_ref[1, 0]
  o_ref[0] = a * _dinv_cols(deg_ref[...]) + b2_ref[...]


_W_SPEC = pl.BlockSpec((D, D), lambda t, i: (0, 0))
_DEG_SPEC = pl.BlockSpec((NC, BR), lambda t, i: (0, t * NB + i))
_ROWS_SPEC = pl.BlockSpec((1, BR, D), lambda t, i: (t, i, 0))
_ACC_SPEC = pl.BlockSpec((NC, 1, BR, D), lambda t, i: (0, t, i, 0))
_BIAS_SPEC = pl.BlockSpec((1, D), lambda t, i: (0, 0))


def _tc1(x_pad, W_lin, W1, degs):
  return pl.pallas_call(
      _tc1_body,
      grid=(T, NB),
      in_specs=[_ROWS_SPEC, _W_SPEC, _W_SPEC, _DEG_SPEC],
      out_specs=_ROWS_SPEC,
      out_shape=jax.ShapeDtypeStruct((T, NP, D), jnp.float32),
  )(x_pad, W_lin, W1, degs)


def _tc2(acc, degs, W2, b1):
  return pl.pallas_call(
      _tc2_body,
      grid=(T, NB),
      in_specs=[_ACC_SPEC, _DEG_SPEC, _W_SPEC, _BIAS_SPEC],
      out_specs=_ROWS_SPEC,
      out_shape=jax.ShapeDtypeStruct((T, NP, D), jnp.float32),
  )(acc, degs, W2, b1)


def _tc3(acc, degs, b2):
  return pl.pallas_call(
      _tc3_body,
      grid=(T, NB),
      in_specs=[_ACC_SPEC, _DEG_SPEC, _BIAS_SPEC],
      out_specs=_ROWS_SPEC,
      out_shape=jax.ShapeDtypeStruct((T, NP, D), jnp.float32),
  )(acc, degs, b2)


@jax.jit
def kernel(edge_index_list, x_list, W_lin, W1, b1, W2, b2):
  ei = edge_index_list.astype(jnp.int32)
  e_last = ei[T - 1]
  loop = jnp.arange(NP, dtype=jnp.int32)
  src_t, dst_t = [], []
  for t in range(T):
    src_t.append(jnp.concatenate([ei[t, 0], e_last[0], loop]) + t * NP)
    dst_t.append(jnp.concatenate([ei[t, 1], e_last[1], loop]))
  src_g = jnp.stack(src_t).reshape(T, NW, NCH, CHUNK).swapaxes(0, 1)
  dst_s = jnp.stack(dst_t)
  dst_l = dst_s.reshape(T, NW, NCH, CHUNK).swapaxes(0, 1)
  dst_deg = (
      dst_s + (jnp.arange(T, dtype=jnp.int32) * NP)[:, None]
  ).reshape(T, NW, NCH, CHUNK).swapaxes(0, 1).reshape(NW, T * NCH, CHUNK)

  degs = _sc_degrees(dst_deg)
  x_pad = jnp.pad(x_list, ((0, 0), (0, NP - N), (0, 0)))
  xs1 = _tc1(x_pad, W_lin, W1, degs)
  acc1 = _sc_gather_scatter(xs1.reshape(T * NP, D), src_g, dst_l)
  xs2 = _tc2(acc1, degs, W2, b1.reshape(1, D))
  acc2 = _sc_gather_scatter(xs2.reshape(T * NP, D), src_g, dst_l)
  out = _tc3(acc2, degs, b2.reshape(1, D))
  return out[:, :N, :]


# speedup vs baseline: 27.7407x; 27.7407x over previous
import jax
import jax.numpy as jnp
from jax import lax
from jax.experimental import pallas as pl
from jax.experimental.pallas import tpu as pltpu
from jax.experimental.pallas import tpu_sc as plsc

T = 3
N = 10000
D = 128
NP = 10240
NC, NS, LANES = 2, 16, 16
NW = NC * NS
EP = 655360
NPAD = EP - 2 * 320000 - NP
PER_TILE = EP // NW
CHUNK = 128
NCH = PER_TILE // CHUNK
GRP = 40
NGRP = NCH // GRP
ROWS_PER_TILE = NP // NS
BR = 1024
NB = NP // BR
DEGC = T * NCH
DEGG = 6


def _mesh():
  return plsc.VectorSubcoreMesh(
      core_axis_name="c", subcore_axis_name="s",
      num_cores=NC, num_subcores=NS)


def _deg_body(dst_hbm, out_hbm, idx_v, ones_v, zer_v, acc_sh, dsem):
  cid = lax.axis_index("c")
  sid = lax.axis_index("s")
  wid = cid * NS + sid
  zslice = (T * NP) // NS

  for j in range(CHUNK // LANES):
    ones_v[pl.ds(j * LANES, LANES)] = jnp.full((LANES,), 1.0, jnp.float32)

  def zbody(j, c):
    zer_v[pl.ds(j * LANES, LANES)] = jnp.zeros((LANES,), jnp.float32)
    return c
  lax.fori_loop(0, zslice // LANES, zbody, 0)

  pltpu.sync_copy(dst_hbm.at[wid], idx_v)
  pltpu.sync_copy(zer_v, acc_sh.at[pl.ds(sid * zslice, zslice)])
  plsc.subcore_barrier()

  def gbody(g, c):
    for u in range(DEGG):
      pltpu.async_copy(ones_v, acc_sh.at[idx_v.at[g * DEGG + u]], dsem,
                       add=True)
    for _ in range(DEGG):
      pltpu.make_async_copy(ones_v, acc_sh.at[idx_v.at[0]], dsem).wait()
    return c
  lax.fori_loop(0, DEGC // DEGG, gbody, 0)
  plsc.subcore_barrier()

  pltpu.sync_copy(acc_sh.at[pl.ds(sid * zslice, zslice)],
                  out_hbm.at[cid, pl.ds(sid * zslice, zslice)])


def _gs_body(table_hbm, src_hbm, dst_hbm, out_hbm,
             src_v, dst_v, bufs, acc_sh, gsem, ssem):
  cid = lax.axis_index("c")
  sid = lax.axis_index("s")
  wid = cid * NS + sid
  row0 = sid * ROWS_PER_TILE

  def start_gather(c, b):
    pltpu.async_copy(table_hbm.at[src_v.at[c]], bufs.at[b], gsem)

  def wait_gather():
    pltpu.make_async_copy(table_hbm.at[src_v.at[0]], bufs.at[0], gsem).wait()

  def start_scatter(c, b):
    pltpu.async_copy(bufs.at[b], acc_sh.at[dst_v.at[c]], ssem, add=True)

  def wait_scatter():
    pltpu.make_async_copy(bufs.at[0], acc_sh.at[dst_v.at[0]], ssem).wait()

  for t in range(T):
    if t > 0:
      pltpu.sync_copy(acc_sh.at[pl.ds(row0, ROWS_PER_TILE)],
                      out_hbm.at[cid, t - 1, pl.ds(row0, ROWS_PER_TILE)])
    def zbody(k, c):
      bufs[0, k >> 3, pl.ds((k & 7) * LANES, LANES)] = jnp.zeros(
          (LANES,), jnp.float32)
      return c
    lax.fori_loop(0, CHUNK * (D // LANES), zbody, 0)
    for z in range(ROWS_PER_TILE // CHUNK):
      pltpu.sync_copy(bufs.at[0], acc_sh.at[pl.ds(row0 + z * CHUNK, CHUNK)])
    plsc.subcore_barrier()

    for g in range(NGRP):
      pltpu.sync_copy(src_hbm.at[wid, t, pl.ds(g * GRP, GRP)], src_v)
      pltpu.sync_copy(dst_hbm.at[wid, t, pl.ds(g * GRP, GRP)], dst_v)
      start_gather(0, 0)

      def cbody(c, carry):
        wait_gather()
        start_scatter(c, lax.rem(c, 2))

        @pl.when(c >= 1)
        def _():
          wait_scatter()

        @pl.when(c + 1 < GRP)
        def _():
          start_gather(c + 1, lax.rem(c + 1, 2))
        return carry
      lax.fori_loop(0, GRP, cbody, 0)
      wait_scatter()
    plsc.subcore_barrier()

  pltpu.sync_copy(acc_sh.at[pl.ds(row0, ROWS_PER_TILE)],
                  out_hbm.at[cid, T - 1, pl.ds(row0, ROWS_PER_TILE)])


@jax.jit
def _sc_degrees(dst_deg):
  return pl.kernel(
      _deg_body,
      out_type=jax.ShapeDtypeStruct((NC, T * NP), jnp.float32),
      mesh=_mesh(),
      scratch_types=[
          pltpu.VMEM((DEGC, CHUNK), jnp.int32),
          pltpu.VMEM((CHUNK,), jnp.float32),
          pltpu.VMEM(((T * NP) // NS,), jnp.float32),
          pltpu.VMEM_SHARED((T * NP,), jnp.float32),
          pltpu.SemaphoreType.DMA,
      ],
  )(dst_deg)


@jax.jit
def _sc_gather_scatter(table, src_g, dst_l):
  return pl.kernel(
      _gs_body,
      out_type=jax.ShapeDtypeStruct((NC, T, NP, D), jnp.float32),
      mesh=_mesh(),
      scratch_types=[
          pltpu.VMEM((GRP, CHUNK), jnp.int32),
          pltpu.VMEM((GRP, CHUNK), jnp.int32),
          pltpu.VMEM((2, CHUNK, D), jnp.float32),
          pltpu.VMEM_SHARED((NP, D), jnp.float32),
          pltpu.SemaphoreType.DMA,
          pltpu.SemaphoreType.DMA,
      ],
  )(table, src_g, dst_l)


def _dinv_cols(degb):
  dtot = degb[0:1, :] + degb[1:2, :]
  dinv = lax.rsqrt(dtot)
  eye = jnp.eye(D, dtype=jnp.float32)
  cols = []
  for s in range(BR // D):
    band = dinv[:, s * D:(s + 1) * D]
    cols.append(lax.dot_general(
        eye, band, (((1,), (1,)), ((), ())),
        preferred_element_type=jnp.float32))
  return jnp.concatenate(cols, axis=0)


def _tc1_body(x_ref, wl_ref, w1_ref, deg_ref, o_ref):
  xb = x_ref[0]
  xw = jnp.dot(jnp.dot(xb, wl_ref[...], preferred_element_type=jnp.float32),
               w1_ref[...], preferred_element_type=jnp.float32)
  o_ref[0] = xw * _dinv_cols(deg_ref[...])


def _tc2_body(acc_ref, deg_ref, w2_ref, b1_ref, o_ref):
  a = acc_ref[0, 0] + acc_ref[1, 0]
  dcol = _dinv_cols(deg_ref[...])
  h = jnp.maximum(a * dcol + b1_ref[...], 0.0)
  o_ref[0] = jnp.dot(h, w2_ref[...],
                     preferred_element_type=jnp.float32) * dcol


def _tc3_body(acc_ref, deg_ref, b2_ref, o_ref):
  a = acc_ref[0, 0] + acc_ref[1, 0]
  o_ref[0] = a * _dinv_cols(deg_ref[...]) + b2_ref[...]


_W_SPEC = pl.BlockSpec((D, D), lambda t, i: (0, 0))
_DEG_SPEC = pl.BlockSpec((NC, BR), lambda t, i: (0, t * NB + i))
_ROWS_SPEC = pl.BlockSpec((1, BR, D), lambda t, i: (t, i, 0))
_ACC_SPEC = pl.BlockSpec((NC, 1, BR, D), lambda t, i: (0, t, i, 0))
_BIAS_SPEC = pl.BlockSpec((1, D), lambda t, i: (0, 0))


def _tc1(x_pad, W_lin, W1, degs):
  return pl.pallas_call(
      _tc1_body,
      grid=(T, NB),
      in_specs=[_ROWS_SPEC, _W_SPEC, _W_SPEC, _DEG_SPEC],
      out_specs=_ROWS_SPEC,
      out_shape=jax.ShapeDtypeStruct((T, NP, D), jnp.float32),
  )(x_pad, W_lin, W1, degs)


def _tc2(acc, degs, W2, b1):
  return pl.pallas_call(
      _tc2_body,
      grid=(T, NB),
      in_specs=[_ACC_SPEC, _DEG_SPEC, _W_SPEC, _BIAS_SPEC],
      out_specs=_ROWS_SPEC,
      out_shape=jax.ShapeDtypeStruct((T, NP, D), jnp.float32),
  )(acc, degs, W2, b1)


def _tc3(acc, degs, b2):
  return pl.pallas_call(
      _tc3_body,
      grid=(T, NB),
      in_specs=[_ACC_SPEC, _DEG_SPEC, _BIAS_SPEC],
      out_specs=_ROWS_SPEC,
      out_shape=jax.ShapeDtypeStruct((T, NP, D), jnp.float32),
  )(acc, degs, b2)


@jax.jit
def kernel(edge_index_list, x_list, W_lin, W1, b1, W2, b2):
  ei = edge_index_list.astype(jnp.int32)
  e_last = ei[T - 1]
  loop = jnp.arange(NP, dtype=jnp.int32)
  padv = N + (jnp.arange(NPAD, dtype=jnp.int32) % (NP - N))
  src_t, dst_t = [], []
  for t in range(T):
    src_t.append(
        jnp.concatenate([ei[t, 0], e_last[0], loop, padv]) + t * NP)
    dst_t.append(jnp.concatenate([ei[t, 1], e_last[1], loop, padv]))
  src_g = jnp.stack(src_t).reshape(T, NW, NCH, CHUNK).swapaxes(0, 1)
  dst_s = jnp.stack(dst_t)
  dst_l = dst_s.reshape(T, NW, NCH, CHUNK).swapaxes(0, 1)
  dst_deg = (
      dst_s + (jnp.arange(T, dtype=jnp.int32) * NP)[:, None]
  ).reshape(T, NW, NCH, CHUNK).swapaxes(0, 1).reshape(NW, DEGC, CHUNK)

  degs = _sc_degrees(dst_deg)
  x_pad = jnp.pad(x_list, ((0, 0), (0, NP - N), (0, 0)))
  xs1 = _tc1(x_pad, W_lin, W1, degs)
  acc1 = _sc_gather_scatter(xs1.reshape(T * NP, D), src_g, dst_l)
  xs2 = _tc2(acc1, degs, W2, b1.reshape(1, D))
  acc2 = _sc_gather_scatter(xs2.reshape(T * NP, D), src_g, dst_l)
  out = _tc3(acc2, degs, b2.reshape(1, D))
  return out[:, :N, :]


# speedup vs baseline: 30.6450x; 1.1047x over previous
import jax
import jax.numpy as jnp
from jax import lax
from jax.experimental import pallas as pl
from jax.experimental.pallas import tpu as pltpu
from jax.experimental.pallas import tpu_sc as plsc

T = 3
T1 = 4
N = 10000
D = 128
NP = 10240
NC, NS, LANES = 2, 16, 16
NW = NC * NS
E = 320000
CHUNK = 128
EP01 = 655360
EP2 = 360448
PAD01 = EP01 - 2 * E - NP
PAD2 = EP2 - E - NP
NCH_T = (EP01 // (NW * CHUNK), EP01 // (NW * CHUNK), EP2 // (NW * CHUNK))
GROUPS_T = ((40, 40, 40, 40), (40, 40, 40, 40), (40, 40, 8))
GRP_MAX = 40
CHT = sum(NCH_T)
ROWS_PER_TILE = NP // NS
BR = 1024
NB = NP // BR
DEGC = 480
DEGG = 6


def _mesh():
  return plsc.VectorSubcoreMesh(
      core_axis_name="c", subcore_axis_name="s",
      num_cores=NC, num_subcores=NS)


def _deg_body(dst_hbm, out_hbm, idx_v, ones_v, zer_v, acc_sh, dsem):
  cid = lax.axis_index("c")
  sid = lax.axis_index("s")
  wid = cid * NS + sid
  zslice = (T * NP) // NS

  for j in range(CHUNK // LANES):
    ones_v[pl.ds(j * LANES, LANES)] = jnp.full((LANES,), 1.0, jnp.float32)

  def zbody(j, c):
    zer_v[pl.ds(j * LANES, LANES)] = jnp.zeros((LANES,), jnp.float32)
    return c
  lax.fori_loop(0, zslice // LANES, zbody, 0)

  pltpu.sync_copy(dst_hbm.at[wid], idx_v)
  pltpu.sync_copy(zer_v, acc_sh.at[pl.ds(sid * zslice, zslice)])
  plsc.subcore_barrier()

  def gbody(g, c):
    for u in range(DEGG):
      pltpu.async_copy(ones_v, acc_sh.at[idx_v.at[g * DEGG + u]], dsem,
                       add=True)
    for _ in range(DEGG):
      pltpu.make_async_copy(ones_v, acc_sh.at[idx_v.at[0]], dsem).wait()
    return c
  lax.fori_loop(0, DEGC // DEGG, gbody, 0)
  plsc.subcore_barrier()

  pltpu.sync_copy(acc_sh.at[pl.ds(sid * zslice, zslice)],
                  out_hbm.at[cid, pl.ds(sid * zslice, zslice)])


def _gs_body(table_hbm, src_hbm, dst_hbm, out_hbm,
             src_v, dst_v, bufs, acc_sh, gsem, ssem):
  cid = lax.axis_index("c")
  sid = lax.axis_index("s")
  wid = cid * NS + sid
  row0 = sid * ROWS_PER_TILE

  def start_gather(c, b):
    pltpu.async_copy(table_hbm.at[src_v.at[c]], bufs.at[b], gsem)

  def wait_gather():
    pltpu.make_async_copy(table_hbm.at[src_v.at[0]], bufs.at[0], gsem).wait()

  def start_scatter(c, b):
    pltpu.async_copy(bufs.at[b], acc_sh.at[dst_v.at[c]], ssem, add=True)

  def wait_scatter():
    pltpu.make_async_copy(bufs.at[0], acc_sh.at[dst_v.at[0]], ssem).wait()

  cbase = 0
  for t in range(T):
    if t > 0:
      pltpu.sync_copy(acc_sh.at[pl.ds(row0, ROWS_PER_TILE)],
                      out_hbm.at[cid, t - 1, pl.ds(row0, ROWS_PER_TILE)])
    def zbody(k, c):
      bufs[0, k >> 3, pl.ds((k & 7) * LANES, LANES)] = jnp.zeros(
          (LANES,), jnp.float32)
      return c
    lax.fori_loop(0, CHUNK * (D // LANES), zbody, 0)
    for z in range(ROWS_PER_TILE // CHUNK):
      pltpu.sync_copy(bufs.at[0], acc_sh.at[pl.ds(row0 + z * CHUNK, CHUNK)])
    plsc.subcore_barrier()

    gbase = cbase
    for grp in GROUPS_T[t]:
      pltpu.sync_copy(src_hbm.at[wid, pl.ds(gbase, grp)],
                      src_v.at[pl.ds(0, grp)])
      pltpu.sync_copy(dst_hbm.at[wid, pl.ds(gbase, grp)],
                      dst_v.at[pl.ds(0, grp)])
      start_gather(0, 0)

      def cbody(c, carry):
        wait_gather()
        start_scatter(c, lax.rem(c, 2))

        @pl.when(c >= 1)
        def _():
          wait_scatter()

        @pl.when(c + 1 < grp)
        def _():
          start_gather(c + 1, lax.rem(c + 1, 2))
        return carry
      lax.fori_loop(0, grp, cbody, 0)
      wait_scatter()
      gbase += grp
    cbase += NCH_T[t]
    plsc.subcore_barrier()

  pltpu.sync_copy(acc_sh.at[pl.ds(row0, ROWS_PER_TILE)],
                  out_hbm.at[cid, T - 1, pl.ds(row0, ROWS_PER_TILE)])


@jax.jit
def _sc_degrees(dst_deg):
  return pl.kernel(
      _deg_body,
      out_type=jax.ShapeDtypeStruct((NC, T * NP), jnp.float32),
      mesh=_mesh(),
      scratch_types=[
          pltpu.VMEM((DEGC, CHUNK), jnp.int32),
          pltpu.VMEM((CHUNK,), jnp.float32),
          pltpu.VMEM(((T * NP) // NS,), jnp.float32),
          pltpu.VMEM_SHARED((T * NP,), jnp.float32),
          pltpu.SemaphoreType.DMA,
      ],
  )(dst_deg)


@jax.jit
def _sc_gather_scatter(table, src_g, dst_l):
  return pl.kernel(
      _gs_body,
      out_type=jax.ShapeDtypeStruct((NC, T, NP, D), jnp.float32),
      mesh=_mesh(),
      scratch_types=[
          pltpu.VMEM((GRP_MAX, CHUNK), jnp.int32),
          pltpu.VMEM((GRP_MAX, CHUNK), jnp.int32),
          pltpu.VMEM((2, CHUNK, D), jnp.float32),
          pltpu.VMEM_SHARED((NP, D), jnp.float32),
          pltpu.SemaphoreType.DMA,
          pltpu.SemaphoreType.DMA,
      ],
  )(table, src_g, dst_l)


def _dinv_cols(degb):
  dtot = degb[0:1, :] + degb[1:2, :]
  dinv = lax.rsqrt(dtot)
  eye = jnp.eye(D, dtype=jnp.float32)
  cols = []
  for s in range(BR // D):
    band = dinv[:, s * D:(s + 1) * D]
    cols.append(lax.dot_general(
        eye, band, (((1,), (1,)), ((), ())),
        preferred_element_type=jnp.float32))
  return jnp.concatenate(cols, axis=0)


def _slot_scale():
  return jnp.where(pl.program_id(0) == T, 2.0, 1.0).astype(jnp.float32)


def _tc1_body(x_ref, wl_ref, w1_ref, deg_ref, o_ref):
  xb = x_ref[0]
  xw = jnp.dot(jnp.dot(xb, wl_ref[...], preferred_element_type=jnp.float32),
               w1_ref[...], preferred_element_type=jnp.float32)
  o_ref[0] = xw * (_dinv_cols(deg_ref[...]) * _slot_scale())


def _tc2_body(acc_ref, deg_ref, w2_ref, b1_ref, o_ref):
  a = acc_ref[0, 0] + acc_ref[1, 0]
  dcol = _dinv_cols(deg_ref[...])
  h = jnp.maximum(a * dcol + b1_ref[...], 0.0)
  o_ref[0] = jnp.dot(h, w2_ref[...],
                     preferred_element_type=jnp.float32) * (
                         dcol * _slot_scale())


def _tc3_body(acc_ref, deg_ref, b2_ref, o_ref):
  a = acc_ref[0, 0] + acc_ref[1, 0]
  o_ref[0] = a * _dinv_cols(deg_ref[...]) + b2_ref[...]


def _tmin(t):
  return jnp.minimum(t, T - 1)


_W_SPEC = pl.BlockSpec((D, D), lambda t, i: (0, 0))
_DEG_SPEC = pl.BlockSpec((NC, BR), lambda t, i: (0, _tmin(t) * NB + i))
_DEG3_SPEC = pl.BlockSpec((NC, BR), lambda t, i: (0, t * NB + i))
_XIN_SPEC = pl.BlockSpec((1, BR, D), lambda t, i: (_tmin(t), i, 0))
_ROWS_SPEC = pl.BlockSpec((1, BR, D), lambda t, i: (t, i, 0))
_ACC_SPEC = pl.BlockSpec((NC, 1, BR, D), lambda t, i: (0, _tmin(t), i, 0))
_ACC3_SPEC = pl.BlockSpec((NC, 1, BR, D), lambda t, i: (0, t, i, 0))
_BIAS_SPEC = pl.BlockSpec((1, D), lambda t, i: (0, 0))


def _tc1(x_pad, W_lin, W1, degs):
  return pl.pallas_call(
      _tc1_body,
      grid=(T1, NB),
      in_specs=[_XIN_SPEC, _W_SPEC, _W_SPEC, _DEG_SPEC],
      out_specs=_ROWS_SPEC,
      out_shape=jax.ShapeDtypeStruct((T1, NP, D), jnp.float32),
  )(x_pad, W_lin, W1, degs)


def _tc2(acc, degs, W2, b1):
  return pl.pallas_call(
      _tc2_body,
      grid=(T1, NB),
      in_specs=[_ACC_SPEC, _DEG_SPEC, _W_SPEC, _BIAS_SPEC],
      out_specs=_ROWS_SPEC,
      out_shape=jax.ShapeDtypeStruct((T1, NP, D), jnp.float32),
  )(acc, degs, W2, b1)


def _tc3(acc, degs, b2):
  return pl.pallas_call(
      _tc3_body,
      grid=(T, NB),
      in_specs=[_ACC3_SPEC, _DEG3_SPEC, _BIAS_SPEC],
      out_specs=_ROWS_SPEC,
      out_shape=jax.ShapeDtypeStruct((T, NP, D), jnp.float32),
  )(acc, degs, b2)


@jax.jit
def kernel(edge_index_list, x_list, W_lin, W1, b1, W2, b2):
  ei = edge_index_list.astype(jnp.int32)
  e_last = ei[T - 1]
  loop = jnp.arange(NP, dtype=jnp.int32)
  pad01 = N + (jnp.arange(PAD01, dtype=jnp.int32) % (NP - N))
  pad2 = N + (jnp.arange(PAD2, dtype=jnp.int32) % (NP - N))
  src_t, dst_t = [], []
  for t in range(2):
    src_t.append(
        (jnp.concatenate([ei[t, 0], e_last[0], loop, pad01]) + t * NP)
        .reshape(NW, NCH_T[t], CHUNK))
    dst_t.append(
        jnp.concatenate([ei[t, 1], e_last[1], loop, pad01])
        .reshape(NW, NCH_T[t], CHUNK))
  src_t.append(
      jnp.concatenate([e_last[0] + (T1 - 1) * NP,
                       loop + 2 * NP, pad2 + 2 * NP])
      .reshape(NW, NCH_T[2], CHUNK))
  dst_t.append(
      jnp.concatenate([e_last[1], loop, pad2]).reshape(NW, NCH_T[2], CHUNK))
  src_g = jnp.concatenate(src_t, axis=1)
  dst_l = jnp.concatenate(dst_t, axis=1)
  dst_deg = jnp.stack(
      [jnp.concatenate([ei[t, 1], e_last[1], loop, pad01]) + t * NP
       for t in range(T)]
  ).reshape(T, NW, NCH_T[0], CHUNK).swapaxes(0, 1).reshape(NW, DEGC, CHUNK)

  degs = _sc_degrees(dst_deg)
  x_pad = jnp.pad(x_list, ((0, 0), (0, NP - N), (0, 0)))
  xs1 = _tc1(x_pad, W_lin, W1, degs)
  acc1 = _sc_gather_scatter(xs1.reshape(T1 * NP, D), src_g, dst_l)
  xs2 = _tc2(acc1, degs, W2, b1.reshape(1, D))
  acc2 = _sc_gather_scatter(xs2.reshape(T1 * NP, D), src_g, dst_l)
  out = _tc3(acc2, degs, b2.reshape(1, D))
  return out[:, :N, :]


# speedup vs baseline: 31.0077x; 1.0118x over previous
import jax
import jax.numpy as jnp
from jax import lax
from jax.experimental import pallas as pl
from jax.experimental.pallas import tpu as pltpu
from jax.experimental.pallas import tpu_sc as plsc

T = 3
T1 = 4
N = 10000
D = 128
NP = 10240
NC, NS, LANES = 2, 16, 16
NW = NC * NS
E = 320000
CHUNK = 128
EP01 = 655360
EP2 = 360448
PAD01 = EP01 - 2 * E - NP
PAD2 = EP2 - E - NP
NCH_T = (EP01 // (NW * CHUNK), EP01 // (NW * CHUNK), EP2 // (NW * CHUNK))
GROUPS_T = ((40, 40, 40, 40), (40, 40, 40, 40), (40, 40, 8))
GRP_MAX = 40
CHT = sum(NCH_T)
ROWS_PER_TILE = NP // NS
BR = 1024
NB = NP // BR
DEGC = 480
DEGG = 6


def _mesh():
  return plsc.VectorSubcoreMesh(
      core_axis_name="c", subcore_axis_name="s",
      num_cores=NC, num_subcores=NS)


def _deg_body(dst_hbm, out_hbm, idx_v, ones_v, zer_v, acc_sh, dsem):
  cid = lax.axis_index("c")
  sid = lax.axis_index("s")
  wid = cid * NS + sid
  zslice = (T * NP) // NS

  for j in range(CHUNK // LANES):
    ones_v[pl.ds(j * LANES, LANES)] = jnp.full((LANES,), 1.0, jnp.float32)

  def zbody(j, c):
    zer_v[pl.ds(j * LANES, LANES)] = jnp.zeros((LANES,), jnp.float32)
    return c
  lax.fori_loop(0, zslice // LANES, zbody, 0)

  pltpu.sync_copy(dst_hbm.at[wid], idx_v)
  pltpu.sync_copy(zer_v, acc_sh.at[pl.ds(sid * zslice, zslice)])
  plsc.subcore_barrier()

  def gbody(g, c):
    for u in range(DEGG):
      pltpu.async_copy(ones_v, acc_sh.at[idx_v.at[g * DEGG + u]], dsem,
                       add=True)
    for _ in range(DEGG):
      pltpu.make_async_copy(ones_v, acc_sh.at[idx_v.at[0]], dsem).wait()
    return c
  lax.fori_loop(0, DEGC // DEGG, gbody, 0)
  plsc.subcore_barrier()

  pltpu.sync_copy(acc_sh.at[pl.ds(sid * zslice, zslice)],
                  out_hbm.at[cid, pl.ds(sid * zslice, zslice)])


def _gs_body(table_hbm, src_hbm, dst_hbm, out_hbm,
             src_v, dst_v, bufs, acc_sh, gsem, ssem):
  cid = lax.axis_index("c")
  sid = lax.axis_index("s")
  wid = cid * NS + sid
  row0 = sid * ROWS_PER_TILE

  def start_gather(c, b):
    pltpu.async_copy(table_hbm.at[src_v.at[c]], bufs.at[b], gsem)

  def wait_gather():
    pltpu.make_async_copy(table_hbm.at[src_v.at[0]], bufs.at[0], gsem).wait()

  def start_scatter(c, b):
    pltpu.async_copy(bufs.at[b], acc_sh.at[dst_v.at[c]], ssem, add=True)

  def wait_scatter():
    pltpu.make_async_copy(bufs.at[0], acc_sh.at[dst_v.at[0]], ssem).wait()

  cbase = 0
  for t in range(T):
    if t > 0:
      pltpu.sync_copy(acc_sh.at[pl.ds(row0, ROWS_PER_TILE)],
                      out_hbm.at[cid, t - 1, pl.ds(row0, ROWS_PER_TILE)])
    def zbody(k, c):
      bufs[0, k >> 3, pl.ds((k & 7) * LANES, LANES)] = jnp.zeros(
          (LANES,), jnp.float32)
      return c
    lax.fori_loop(0, CHUNK * (D // LANES), zbody, 0)
    for z in range(ROWS_PER_TILE // CHUNK):
      pltpu.sync_copy(bufs.at[0], acc_sh.at[pl.ds(row0 + z * CHUNK, CHUNK)])
    plsc.subcore_barrier()

    gbase = cbase
    for grp in GROUPS_T[t]:
      pltpu.sync_copy(src_hbm.at[wid, pl.ds(gbase, grp)],
                      src_v.at[pl.ds(0, grp)])
      pltpu.sync_copy(dst_hbm.at[wid, pl.ds(gbase, grp)],
                      dst_v.at[pl.ds(0, grp)])
      start_gather(0, 0)

      def cbody(c, carry):
        wait_gather()
        start_scatter(c, lax.rem(c, 2))

        @pl.when(c >= 1)
        def _():
          wait_scatter()

        @pl.when(c + 1 < grp)
        def _():
          start_gather(c + 1, lax.rem(c + 1, 2))
        return carry
      lax.fori_loop(0, grp, cbody, 0)
      wait_scatter()
      gbase += grp
    cbase += NCH_T[t]
    plsc.subcore_barrier()

  pltpu.sync_copy(acc_sh.at[pl.ds(row0, ROWS_PER_TILE)],
                  out_hbm.at[cid, T - 1, pl.ds(row0, ROWS_PER_TILE)])


@jax.jit
def _sc_degrees(dst_deg):
  return pl.kernel(
      _deg_body,
      out_type=jax.ShapeDtypeStruct((NC, T * NP), jnp.float32),
      mesh=_mesh(),
      scratch_types=[
          pltpu.VMEM((DEGC, CHUNK), jnp.int32),
          pltpu.VMEM((CHUNK,), jnp.float32),
          pltpu.VMEM(((T * NP) // NS,), jnp.float32),
          pltpu.VMEM_SHARED((T * NP,), jnp.float32),
          pltpu.SemaphoreType.DMA,
      ],
  )(dst_deg)


@jax.jit
def _sc_gather_scatter(table, src_g, dst_l):
  return pl.kernel(
      _gs_body,
      out_type=jax.ShapeDtypeStruct((NC, T, NP, D), jnp.float32),
      mesh=_mesh(),
      scratch_types=[
          pltpu.VMEM((GRP_MAX, CHUNK), jnp.int32),
          pltpu.VMEM((GRP_MAX, CHUNK), jnp.int32),
          pltpu.VMEM((2, CHUNK, D), jnp.float32),
          pltpu.VMEM_SHARED((NP, D), jnp.float32),
          pltpu.SemaphoreType.DMA,
          pltpu.SemaphoreType.DMA,
      ],
  )(table, src_g, dst_l)


def _dinv_cols(degb):
  dtot = degb[0:1, :] + degb[1:2, :]
  dinv = lax.rsqrt(dtot)
  eye = jnp.eye(D, dtype=jnp.float32)
  cols = []
  for s in range(BR // D):
    band = dinv[:, s * D:(s + 1) * D]
    cols.append(lax.dot_general(
        eye, band, (((1,), (1,)), ((), ())),
        preferred_element_type=jnp.float32))
  return jnp.concatenate(cols, axis=0)


def _slot_scale():
  return jnp.where(pl.program_id(0) == T, 2.0, 1.0).astype(jnp.float32)


def _tc1_body(x_ref, wl_ref, w1_ref, deg_ref, o_ref):
  xb = x_ref[0]
  xw = jnp.dot(jnp.dot(xb, wl_ref[...], preferred_element_type=jnp.float32),
               w1_ref[...], preferred_element_type=jnp.float32)
  o_ref[0] = xw * (_dinv_cols(deg_ref[...]) * _slot_scale())


def _tc2_body(acc_ref, deg_ref, w2_ref, b1_ref, o_ref):
  a = acc_ref[0, 0] + acc_ref[1, 0]
  dcol = _dinv_cols(deg_ref[...])
  h = jnp.maximum(a * dcol + b1_ref[...], 0.0)
  o_ref[0] = jnp.dot(h, w2_ref[...],
                     preferred_element_type=jnp.float32) * (
                         dcol * _slot_scale())


def _tc3_body(acc_ref, deg_ref, b2_ref, o_ref):
  a = acc_ref[0, 0] + acc_ref[1, 0]
  o_ref[0] = a * _dinv_cols(deg_ref[...]) + b2_ref[...]


def _tmin(t):
  return jnp.minimum(t, T - 1)


_W_SPEC = pl.BlockSpec((D, D), lambda t, i: (0, 0))
_DEG_SPEC = pl.BlockSpec((NC, BR), lambda t, i: (0, _tmin(t) * NB + i))
_DEG3_SPEC = pl.BlockSpec((NC, BR), lambda t, i: (0, t * NB + i))
_XIN_SPEC = pl.BlockSpec((1, BR, D), lambda t, i: (_tmin(t), i, 0))
_ROWS_SPEC = pl.BlockSpec((1, BR, D), lambda t, i: (t, i, 0))
_ACC_SPEC = pl.BlockSpec((NC, 1, BR, D), lambda t, i: (0, _tmin(t), i, 0))
_ACC3_SPEC = pl.BlockSpec((NC, 1, BR, D), lambda t, i: (0, t, i, 0))
_BIAS_SPEC = pl.BlockSpec((1, D), lambda t, i: (0, 0))


def _tc1(x_list, W_lin, W1, degs):
  return pl.pallas_call(
      _tc1_body,
      grid=(T1, NB),
      in_specs=[_XIN_SPEC, _W_SPEC, _W_SPEC, _DEG_SPEC],
      out_specs=_ROWS_SPEC,
      out_shape=jax.ShapeDtypeStruct((T1, NP, D), jnp.float32),
  )(x_list, W_lin, W1, degs)


def _tc2(acc, degs, W2, b1):
  return pl.pallas_call(
      _tc2_body,
      grid=(T1, NB),
      in_specs=[_ACC_SPEC, _DEG_SPEC, _W_SPEC, _BIAS_SPEC],
      out_specs=_ROWS_SPEC,
      out_shape=jax.ShapeDtypeStruct((T1, NP, D), jnp.float32),
  )(acc, degs, W2, b1)


def _tc3(acc, degs, b2):
  return pl.pallas_call(
      _tc3_body,
      grid=(T, NB),
      in_specs=[_ACC3_SPEC, _DEG3_SPEC, _BIAS_SPEC],
      out_specs=_ROWS_SPEC,
      out_shape=jax.ShapeDtypeStruct((T, N, D), jnp.float32),
  )(acc, degs, b2)


@jax.jit
def kernel(edge_index_list, x_list, W_lin, W1, b1, W2, b2):
  ei = edge_index_list.astype(jnp.int32)
  e_last = ei[T - 1]
  loop = jnp.arange(NP, dtype=jnp.int32)
  pad01 = N + (jnp.arange(PAD01, dtype=jnp.int32) % (NP - N))
  pad2 = N + (jnp.arange(PAD2, dtype=jnp.int32) % (NP - N))
  src_t, dst_t = [], []
  for t in range(2):
    src_t.append(
        (jnp.concatenate([ei[t, 0], e_last[0], loop, pad01]) + t * NP)
        .reshape(NW, NCH_T[t], CHUNK))
    dst_t.append(
        jnp.concatenate([ei[t, 1], e_last[1], loop, pad01])
        .reshape(NW, NCH_T[t], CHUNK))
  src_t.append(
      jnp.concatenate([e_last[0] + (T1 - 1) * NP,
                       loop + 2 * NP, pad2 + 2 * NP])
      .reshape(NW, NCH_T[2], CHUNK))
  dst_t.append(
      jnp.concatenate([e_last[1], loop, pad2]).reshape(NW, NCH_T[2], CHUNK))
  src_g = jnp.concatenate(src_t, axis=1)
  dst_l = jnp.concatenate(dst_t, axis=1)
  dst_deg = jnp.stack(
      [jnp.concatenate([ei[t, 1], e_last[1], loop, pad01]) + t * NP
       for t in range(T)]
  ).reshape(T, NW, NCH_T[0], CHUNK).swapaxes(0, 1).reshape(NW, DEGC, CHUNK)

  degs = _sc_degrees(dst_deg)
  xs1 = _tc1(x_list, W_lin, W1, degs)
  acc1 = _sc_gather_scatter(xs1.reshape(T1 * NP, D), src_g, dst_l)
  xs2 = _tc2(acc1, degs, W2, b1.reshape(1, D))
  acc2 = _sc_gather_scatter(xs2.reshape(T1 * NP, D), src_g, dst_l)
  return _tc3(acc2, degs, b2.reshape(1, D))
